# baseline (speedup 1.0000x reference)
import functools
import jax
import jax.numpy as jnp
from jax import lax
from jax.experimental import pallas as pl
from jax.experimental.pallas import tpu as pltpu
from jax.experimental.pallas import tpu_sc as plsc

N = 10000
E = 320000
F_IN = 128
T = 12
OUT = 32
HORIZON = 12

NSC = 2
NTILE = 16
E_SC = E // NSC
EB = 128
NBLK_SC = E_SC // EB

CHUNK_T = 2
CW = 64 * CHUNK_T
NCH = T // CHUNK_T
N_PAD = 10240
ROWS_TILE = N_PAD // NTILE

_mesh = plsc.VectorSubcoreMesh(core_axis_name="c", subcore_axis_name="s")


def _ntrip(s):
    return jnp.where(s < NBLK_SC % NTILE, NBLK_SC // NTILE + 1,
                     NBLK_SC // NTILE)


@functools.partial(
    pl.kernel,
    out_type=jax.ShapeDtypeStruct((NSC, N_PAD), jnp.float32),
    mesh=_mesh,
    scratch_types=[
        pltpu.VMEM((EB,), jnp.int32),
        pltpu.VMEM((EB,), jnp.float32),
        pltpu.VMEM((ROWS_TILE,), jnp.float32),
        pltpu.VMEM_SHARED((N_PAD,), jnp.float32),
    ],
)
def _deg_kernel(dst_hbm, ew_hbm, out_hbm, idx_v, val_v, zero_v, acc_sh):
    c = lax.axis_index("c")
    s = lax.axis_index("s")

    for j in range(ROWS_TILE // 16):
        zero_v[pl.ds(16 * j, 16)] = jnp.zeros((16,), jnp.float32)

    pltpu.sync_copy(zero_v, acc_sh.at[pl.ds(s * ROWS_TILE, ROWS_TILE)])
    plsc.subcore_barrier()

    def blk(b, carry):
        eoff = pl.multiple_of(c * E_SC + (b * NTILE + s) * EB, EB)
        pltpu.sync_copy(dst_hbm.at[pl.ds(eoff, EB)], idx_v)
        pltpu.sync_copy(ew_hbm.at[pl.ds(eoff, EB)], val_v)
        pltpu.sync_copy(val_v, acc_sh.at[idx_v], add=True)
        return carry

    lax.fori_loop(0, _ntrip(s), blk, 0)
    plsc.subcore_barrier()
    pltpu.sync_copy(acc_sh.at[pl.ds(s * ROWS_TILE, ROWS_TILE)],
                    out_hbm.at[c, pl.ds(s * ROWS_TILE, ROWS_TILE)])


NBF = 78
NSTAGE = 80


@functools.partial(
    pl.kernel,
    out_type=jax.ShapeDtypeStruct((NSC, NCH, N_PAD, CW), jnp.float32),
    mesh=_mesh,
    scratch_types=[
        pltpu.VMEM((NSTAGE, EB), jnp.int32),
        pltpu.VMEM((NSTAGE,), jnp.int32),
        pltpu.VMEM((EB, CW), jnp.float32),
        pltpu.VMEM((EB, CW), jnp.float32),
        pltpu.VMEM((EB,), jnp.int32),
        pltpu.VMEM((EB,), jnp.int32),
        pltpu.VMEM((EB,), jnp.float32),
        pltpu.VMEM((EB,), jnp.float32),
        pltpu.VMEM_SHARED((N_PAD, CW), jnp.float32),
        pltpu.SemaphoreType.DMA,
        pltpu.SemaphoreType.DMA,
        pltpu.SemaphoreType.DMA,
        pltpu.SemaphoreType.DMA,
    ],
)
def _msg_kernel(src_hbm, dst_hbm, ew_hbm, *rest):
    v_tables = rest[:NCH]
    p_hbm = rest[NCH]
    (st_src, bidx_v, rows0, rows1, dst0, dst1, ew0, ew1, acc_sh,
     sem0, sem1, sem_s0, sem_s1) = rest[NCH + 1:]
    c = lax.axis_index("c")
    s = lax.axis_index("s")
    ntrip = NBF + jnp.where(s < NBLK_SC % NTILE, 1, 0)

    def bi(g, carry):
        bvec = g * 16 + lax.broadcasted_iota(jnp.int32, (16,), 0)
        raw = bvec * NTILE + s
        bidx_v[pl.ds(pl.multiple_of(g * 16, 16), 16)] = (
            jnp.minimum(raw, NBLK_SC - 1) + c * NBLK_SC)
        return carry

    lax.fori_loop(0, NSTAGE // 16, bi, 0)

    pltpu.async_copy(src_hbm.at[bidx_v], st_src, sem0).wait()

    bufs = (rows0, rows1)
    dsts = (dst0, dst1)
    ews = (ew0, ew1)
    sems = (sem0, sem1)
    ssems = (sem_s0, sem_s1)

    for ch in range(NCH):
        vt = v_tables[ch]

        def zrow(i, carry):
            for j in range(CW // 16):
                rows0[i, pl.ds(16 * j, 16)] = jnp.zeros((16,), jnp.float32)
            return carry

        lax.fori_loop(0, EB, zrow, 0)
        for m in range(ROWS_TILE // EB):
            pltpu.sync_copy(rows0,
                            acc_sh.at[pl.ds(s * ROWS_TILE + EB * m, EB)])
        plsc.subcore_barrier()

        def fetch_descs(b, par):
            row = c * NBLK_SC + b * NTILE + s
            return (
                pltpu.make_async_copy(vt.at[st_src.at[b]], bufs[par],
                                      sems[par]),
                pltpu.make_async_copy(dst_hbm.at[row], dsts[par], sems[par]),
                pltpu.make_async_copy(ew_hbm.at[row], ews[par], sems[par]),
            )

        def start_fetch(b, par):
            for d in fetch_descs(b, par):
                d.start()

        def wait_fetch(b, par):
            for d in fetch_descs(b, par):
                d.wait()

        def scale_rows(b, par):
            buf = bufs[par]
            ew_v = ews[par]

            def scale(g, carry2):
                ew_vec = ew_v[pl.ds(pl.multiple_of(g * 16, 16), 16)]
                for l in range(16):
                    i = g * 16 + l
                    ew_b = lax.broadcast_in_dim(ew_vec[l], (16,), ())
                    for j in range(CW // 16):
                        buf[i, pl.ds(16 * j, 16)] = (
                            buf[i, pl.ds(16 * j, 16)] * ew_b)
                return carry2

            lax.fori_loop(0, EB // 16, scale, 0)

        def start_scatter(par):
            pltpu.async_copy(bufs[par], acc_sh.at[dsts[par]], ssems[par],
                             add=True)

        def wait_scatter(par):
            pltpu.make_async_copy(bufs[par], acc_sh.at[dsts[par]],
                                  ssems[par]).wait()

        start_fetch(0, 0)
        start_fetch(1, 1)

        def piter(it, carry):
            for par in range(2):
                b = 2 * it + par
                wait_fetch(b, par)
                scale_rows(b, par)
                start_scatter(par)
                other = 1 - par
                bprev = b - 1

                @pl.when(bprev >= 0)
                def _():
                    wait_scatter(other)

                @pl.when((bprev >= 0) & (bprev + 2 < ntrip))
                def _():
                    start_fetch(bprev + 2, other)
            return carry

        lax.fori_loop(0, NBF // 2, piter, 0)

        @pl.when(ntrip > NBF)
        def _():
            wait_fetch(NBF, 0)
            scale_rows(NBF, 0)
            start_scatter(0)

        wait_scatter(1)

        @pl.when(ntrip > NBF)
        def _():
            wait_scatter(0)

        plsc.subcore_barrier()
        pltpu.sync_copy(acc_sh.at[pl.ds(s * ROWS_TILE, ROWS_TILE)],
                        p_hbm.at[c, ch, pl.ds(s * ROWS_TILE, ROWS_TILE)])
        plsc.subcore_barrier()


NT = 512
GRID1 = N_PAD // NT


def _tc1_body(xt_ref, w2_ref, degp_ref, *out_refs):
    v_refs = out_refs[:NCH]
    dinv_ref = out_refs[NCH]
    deg = degp_ref[0, :] + degp_ref[1, :] + 1.0
    dinv = lax.rsqrt(deg)
    dinv_ref[...] = dinv
    xb = xt_ref[...]
    xm = jnp.reshape(xb, (NT * T, F_IN))
    u = jnp.dot(xm, w2_ref[...], preferred_element_type=jnp.float32)
    v = jnp.reshape(u, (NT, T, 64)) * dinv[:, None, None]
    for ch in range(NCH):
        v_refs[ch][...] = jnp.reshape(
            v[:, CHUNK_T * ch:CHUNK_T * (ch + 1), :], (NT, CW))


_tc1 = pl.pallas_call(
    _tc1_body,
    grid=(GRID1,),
    in_specs=[
        pl.BlockSpec((NT, T, F_IN), lambda i: (i, 0, 0)),
        pl.BlockSpec((F_IN, 64), lambda i: (0, 0)),
        pl.BlockSpec((NSC, NT), lambda i: (0, i)),
    ],
    out_specs=[pl.BlockSpec((NT, CW), lambda i: (i, 0))] * NCH
    + [pl.BlockSpec((NT,), lambda i: (i,))],
    out_shape=[jax.ShapeDtypeStruct((N, CW), jnp.float32)] * NCH
    + [jax.ShapeDtypeStruct((N,), jnp.float32)],
)


def _tc2_body(*args):
    p_ref = args[0]
    v_refs = args[1:1 + NCH]
    dinv_ref, bz_ref, bh_ref, probs_ref, linw_ref, linb_ref, out_ref = \
        args[1 + NCH:]
    dv = dinv_ref[...]
    pv = probs_ref[...]
    hacc = jnp.zeros((NT, OUT), jnp.float32)
    for ch in range(NCH):
        sfull = p_ref[0, ch] + p_ref[1, ch] + v_refs[ch][...]
        g = sfull * dv[:, None]
        for k in range(CHUNK_T):
            t = CHUNK_T * ch + k
            z = g[:, 64 * k:64 * k + OUT] + bz_ref[0]
            h = g[:, 64 * k + OUT:64 * k + 64] + bh_ref[0]
            zz = jax.nn.sigmoid(z)
            ht = jnp.tanh(h)
            hacc = hacc + pv[0, t] * (1.0 - zz) * ht
    out_ref[...] = (jnp.dot(jnp.maximum(hacc, 0.0), linw_ref[...],
                            preferred_element_type=jnp.float32)
                    + linb_ref[0])


_tc2 = pl.pallas_call(
    _tc2_body,
    grid=(GRID1,),
    in_specs=[pl.BlockSpec((NSC, NCH, NT, CW), lambda i: (0, 0, i, 0))]
    + [pl.BlockSpec((NT, CW), lambda i: (i, 0))] * NCH
    + [
        pl.BlockSpec((NT,), lambda i: (i,)),
        pl.BlockSpec((1, OUT), lambda i: (0, 0)),
        pl.BlockSpec((1, OUT), lambda i: (0, 0)),
        pl.BlockSpec((1, T), lambda i: (0, 0)),
        pl.BlockSpec((OUT, HORIZON), lambda i: (0, 0)),
        pl.BlockSpec((1, HORIZON), lambda i: (0, 0)),
    ],
    out_specs=pl.BlockSpec((NT, HORIZON), lambda i: (i, 0)),
    out_shape=jax.ShapeDtypeStruct((N, HORIZON), jnp.float32),
)


def kernel(x, edge_index, edge_weight, W_z, b_z, W_r, b_r, W_h, b_h,
           Lz_W, Lz_b, Lr_W, Lr_b, Lh_W, Lh_b, att, lin_W, lin_b):
    src = edge_index[0]
    dst = edge_index[1]

    w2 = jnp.concatenate([W_z @ Lz_W[:OUT], W_h @ Lh_W[:OUT]], axis=1)
    bz = (b_z @ Lz_W[:OUT] + Lz_b).reshape(1, OUT)
    bh = (b_h @ Lh_W[:OUT] + Lh_b).reshape(1, OUT)
    probs = jax.nn.softmax(att).reshape(1, T)
    xt = jnp.transpose(x, (0, 2, 1))

    degp = _deg_kernel(dst, edge_weight)
    tc1_out = _tc1(xt, w2, degp)
    vs = tc1_out[:NCH]
    dinv = tc1_out[NCH]
    p = _msg_kernel(src.reshape(-1, EB), dst.reshape(-1, EB),
                    edge_weight.reshape(-1, EB), *vs)
    return _tc2(p, *vs, dinv, bz, bh, probs, lin_W,
                lin_b.reshape(1, HORIZON))


# speedup vs baseline: 1.2676x; 1.2676x over previous
import functools
import jax
import jax.numpy as jnp
from jax import lax
from jax.experimental import pallas as pl
from jax.experimental.pallas import tpu as pltpu
from jax.experimental.pallas import tpu_sc as plsc

N = 10000
E = 320000
F_IN = 128
T = 12
OUT = 32
HORIZON = 12

NSC = 2
NTILE = 16
E_SC = E // NSC
EB = 128
NBLK_SC = E_SC // EB

CHUNK_T = 2
CW = 64 * CHUNK_T
NCH = T // CHUNK_T
N_PAD = 10240
ROWS_TILE = N_PAD // NTILE

_mesh = plsc.VectorSubcoreMesh(core_axis_name="c", subcore_axis_name="s")


def _ntrip(s):
    return jnp.where(s < NBLK_SC % NTILE, NBLK_SC // NTILE + 1,
                     NBLK_SC // NTILE)


@functools.partial(
    pl.kernel,
    out_type=jax.ShapeDtypeStruct((NSC, N_PAD), jnp.float32),
    mesh=_mesh,
    scratch_types=[
        pltpu.VMEM((EB,), jnp.int32),
        pltpu.VMEM((EB,), jnp.float32),
        pltpu.VMEM((ROWS_TILE,), jnp.float32),
        pltpu.VMEM_SHARED((N_PAD,), jnp.float32),
    ],
)
def _deg_kernel(dst_hbm, ew_hbm, out_hbm, idx_v, val_v, zero_v, acc_sh):
    c = lax.axis_index("c")
    s = lax.axis_index("s")

    for j in range(ROWS_TILE // 16):
        zero_v[pl.ds(16 * j, 16)] = jnp.zeros((16,), jnp.float32)

    pltpu.sync_copy(zero_v, acc_sh.at[pl.ds(s * ROWS_TILE, ROWS_TILE)])
    plsc.subcore_barrier()

    def blk(b, carry):
        eoff = pl.multiple_of(c * E_SC + (b * NTILE + s) * EB, EB)
        pltpu.sync_copy(dst_hbm.at[pl.ds(eoff, EB)], idx_v)
        pltpu.sync_copy(ew_hbm.at[pl.ds(eoff, EB)], val_v)
        pltpu.sync_copy(val_v, acc_sh.at[idx_v], add=True)
        return carry

    lax.fori_loop(0, _ntrip(s), blk, 0)
    plsc.subcore_barrier()
    pltpu.sync_copy(acc_sh.at[pl.ds(s * ROWS_TILE, ROWS_TILE)],
                    out_hbm.at[c, pl.ds(s * ROWS_TILE, ROWS_TILE)])


NBF = 78
NSTAGE = 80


@functools.partial(
    pl.kernel,
    out_type=jax.ShapeDtypeStruct((NSC, NCH, N_PAD, CW), jnp.float32),
    mesh=_mesh,
    scratch_types=[
        pltpu.VMEM((NSTAGE, EB), jnp.int32),
        pltpu.VMEM((NSTAGE,), jnp.int32),
        pltpu.VMEM((EB, CW), jnp.float32),
        pltpu.VMEM((EB, CW), jnp.float32),
        pltpu.VMEM((EB,), jnp.int32),
        pltpu.VMEM((EB,), jnp.int32),
        pltpu.VMEM((EB,), jnp.float32),
        pltpu.VMEM((EB,), jnp.float32),
        pltpu.VMEM_SHARED((N_PAD, CW), jnp.float32),
        pltpu.SemaphoreType.DMA,
        pltpu.SemaphoreType.DMA,
    ],
)
def _msg_kernel(src_hbm, dst_hbm, ew_hbm, *rest):
    v_tables = rest[:NCH]
    p_hbm = rest[NCH]
    (st_src, bidx_v, rows0, rows1, dst0, dst1, ew0, ew1, acc_sh,
     sem0, sem1) = rest[NCH + 1:]
    c = lax.axis_index("c")
    s = lax.axis_index("s")
    ntrip = NBF + jnp.where(s < NBLK_SC % NTILE, 1, 0)

    def bi(g, carry):
        bvec = g * 16 + lax.broadcasted_iota(jnp.int32, (16,), 0)
        raw = bvec * NTILE + s
        bidx_v[pl.ds(pl.multiple_of(g * 16, 16), 16)] = (
            jnp.minimum(raw, NBLK_SC - 1) + c * NBLK_SC)
        return carry

    lax.fori_loop(0, NSTAGE // 16, bi, 0)

    pltpu.async_copy(src_hbm.at[bidx_v], st_src, sem0).wait()

    bufs = (rows0, rows1)
    dsts = (dst0, dst1)
    ews = (ew0, ew1)
    sems = (sem0, sem1)

    for ch in range(NCH):
        vt = v_tables[ch]

        def zrow(i, carry):
            for j in range(CW // 16):
                rows0[i, pl.ds(16 * j, 16)] = jnp.zeros((16,), jnp.float32)
            return carry

        lax.fori_loop(0, EB, zrow, 0)
        for m in range(ROWS_TILE // EB):
            pltpu.sync_copy(rows0,
                            acc_sh.at[pl.ds(s * ROWS_TILE + EB * m, EB)])
        plsc.subcore_barrier()

        def fetch_descs(b, par):
            row = c * NBLK_SC + b * NTILE + s
            return (
                pltpu.make_async_copy(vt.at[st_src.at[b]], bufs[par],
                                      sems[par]),
                pltpu.make_async_copy(dst_hbm.at[row], dsts[par], sems[par]),
                pltpu.make_async_copy(ew_hbm.at[row], ews[par], sems[par]),
            )

        def start_fetch(b, par):
            for d in fetch_descs(b, par):
                d.start()

        def wait_fetch(b, par):
            for d in fetch_descs(b, par):
                d.wait()

        def process(b, par):
            buf = bufs[par]
            ew_v = ews[par]

            def scale(g, carry2):
                ew_vec = ew_v[pl.ds(pl.multiple_of(g * 16, 16), 16)]
                for l in range(16):
                    i = g * 16 + l
                    ew_b = lax.broadcast_in_dim(ew_vec[l], (16,), ())
                    for j in range(CW // 16):
                        buf[i, pl.ds(16 * j, 16)] = (
                            buf[i, pl.ds(16 * j, 16)] * ew_b)
                return carry2

            lax.fori_loop(0, EB // 16, scale, 0)
            pltpu.sync_copy(buf, acc_sh.at[dsts[par]], add=True)

        start_fetch(0, 0)
        start_fetch(1, 1)

        def piter(it, carry):
            for par in range(2):
                b = 2 * it + par
                wait_fetch(b, par)
                process(b, par)

                @pl.when(b + 2 < ntrip)
                def _():
                    start_fetch(b + 2, par)
            return carry

        lax.fori_loop(0, NBF // 2, piter, 0)

        @pl.when(ntrip > NBF)
        def _():
            wait_fetch(NBF, 0)
            process(NBF, 0)

        plsc.subcore_barrier()
        pltpu.sync_copy(acc_sh.at[pl.ds(s * ROWS_TILE, ROWS_TILE)],
                        p_hbm.at[c, ch, pl.ds(s * ROWS_TILE, ROWS_TILE)])
        plsc.subcore_barrier()


NT = 512
GRID1 = N_PAD // NT


def _tc1_body(xt_ref, w2_ref, degp_ref, *out_refs):
    v_refs = out_refs[:NCH]
    dinv_ref = out_refs[NCH]
    deg = degp_ref[0, :] + degp_ref[1, :] + 1.0
    dinv = lax.rsqrt(deg)
    dinv_ref[...] = dinv
    xb = xt_ref[...]
    xm = jnp.reshape(xb, (NT * T, F_IN))
    u = jnp.dot(xm, w2_ref[...], preferred_element_type=jnp.float32)
    v = jnp.reshape(u, (NT, T, 64)) * dinv[:, None, None]
    for ch in range(NCH):
        v_refs[ch][...] = jnp.reshape(
            v[:, CHUNK_T * ch:CHUNK_T * (ch + 1), :], (NT, CW))


_tc1 = pl.pallas_call(
    _tc1_body,
    grid=(GRID1,),
    in_specs=[
        pl.BlockSpec((NT, T, F_IN), lambda i: (i, 0, 0)),
        pl.BlockSpec((F_IN, 64), lambda i: (0, 0)),
        pl.BlockSpec((NSC, NT), lambda i: (0, i)),
    ],
    out_specs=[pl.BlockSpec((NT, CW), lambda i: (i, 0))] * NCH
    + [pl.BlockSpec((NT,), lambda i: (i,))],
    out_shape=[jax.ShapeDtypeStruct((N, CW), jnp.float32)] * NCH
    + [jax.ShapeDtypeStruct((N,), jnp.float32)],
)


def _tc2_body(*args):
    p_ref = args[0]
    v_refs = args[1:1 + NCH]
    dinv_ref, bz_ref, bh_ref, probs_ref, linw_ref, linb_ref, out_ref = \
        args[1 + NCH:]
    dv = dinv_ref[...]
    pv = probs_ref[...]
    hacc = jnp.zeros((NT, OUT), jnp.float32)
    for ch in range(NCH):
        sfull = p_ref[0, ch] + p_ref[1, ch] + v_refs[ch][...]
        g = sfull * dv[:, None]
        for k in range(CHUNK_T):
            t = CHUNK_T * ch + k
            z = g[:, 64 * k:64 * k + OUT] + bz_ref[0]
            h = g[:, 64 * k + OUT:64 * k + 64] + bh_ref[0]
            zz = jax.nn.sigmoid(z)
            ht = jnp.tanh(h)
            hacc = hacc + pv[0, t] * (1.0 - zz) * ht
    out_ref[...] = (jnp.dot(jnp.maximum(hacc, 0.0), linw_ref[...],
                            preferred_element_type=jnp.float32)
                    + linb_ref[0])


_tc2 = pl.pallas_call(
    _tc2_body,
    grid=(GRID1,),
    in_specs=[pl.BlockSpec((NSC, NCH, NT, CW), lambda i: (0, 0, i, 0))]
    + [pl.BlockSpec((NT, CW), lambda i: (i, 0))] * NCH
    + [
        pl.BlockSpec((NT,), lambda i: (i,)),
        pl.BlockSpec((1, OUT), lambda i: (0, 0)),
        pl.BlockSpec((1, OUT), lambda i: (0, 0)),
        pl.BlockSpec((1, T), lambda i: (0, 0)),
        pl.BlockSpec((OUT, HORIZON), lambda i: (0, 0)),
        pl.BlockSpec((1, HORIZON), lambda i: (0, 0)),
    ],
    out_specs=pl.BlockSpec((NT, HORIZON), lambda i: (i, 0)),
    out_shape=jax.ShapeDtypeStruct((N, HORIZON), jnp.float32),
)


def kernel(x, edge_index, edge_weight, W_z, b_z, W_r, b_r, W_h, b_h,
           Lz_W, Lz_b, Lr_W, Lr_b, Lh_W, Lh_b, att, lin_W, lin_b):
    src = edge_index[0]
    dst = edge_index[1]

    w2 = jnp.concatenate([W_z @ Lz_W[:OUT], W_h @ Lh_W[:OUT]], axis=1)
    bz = (b_z @ Lz_W[:OUT] + Lz_b).reshape(1, OUT)
    bh = (b_h @ Lh_W[:OUT] + Lh_b).reshape(1, OUT)
    probs = jax.nn.softmax(att).reshape(1, T)
    xt = jnp.transpose(x, (0, 2, 1))

    degp = _deg_kernel(dst, edge_weight)
    tc1_out = _tc1(xt, w2, degp)
    vs = tc1_out[:NCH]
    dinv = tc1_out[NCH]
    p = _msg_kernel(src.reshape(-1, EB), dst.reshape(-1, EB),
                    edge_weight.reshape(-1, EB), *vs)
    return _tc2(p, *vs, dinv, bz, bh, probs, lin_W,
                lin_b.reshape(1, HORIZON))
